# baseline (speedup 1.0000x reference)
import functools
import jax
import jax.numpy as jnp
from jax import lax
from jax.experimental import pallas as pl
from jax.experimental.pallas import tpu as pltpu
from jax.experimental.pallas import tpu_sc as plsc

_B, _N, _F, _D, _S = 256, 64, 9, 768, 16
_NG, _NS = 2, 2
_NA, _ND = 512, 512
_K = _NA + _ND
_NC, _NSUB = 2, 16
_NW = _NC * _NSUB
_RW = (_B * _N) // _NW
_CH = 8
_G = 10 * _CH

_BB = 16
_R = _BB * _N


def _sc_atom(idsf, tok, tab):
    mesh = plsc.VectorSubcoreMesh(core_axis_name="c", subcore_axis_name="s")

    @functools.partial(
        pl.kernel,
        mesh=mesh,
        out_type=jax.ShapeDtypeStruct((_B * _N, _D), jnp.float32),
        scratch_types=[
            pltpu.VMEM((_G,), jnp.int32),
            pltpu.VMEM((_G, _D), jnp.float32),
            pltpu.VMEM((_CH, _D), jnp.float32),
            pltpu.SemaphoreType.DMA,
        ],
    )
    def k(idsf_hbm, tok_hbm, tab_hbm, out_hbm, idx_v, rows_v, acc_v, sem):
        cid = lax.axis_index("c")
        sid = lax.axis_index("s")
        wid = sid * _NC + cid
        base = wid * _RW

        def chunk(ci, carry):
            rbase = base + ci * _CH
            pltpu.sync_copy(idsf_hbm.at[pl.ds(rbase * 10, _G)], idx_v)
            pltpu.async_copy(tab_hbm.at[idx_v], rows_v, sem).wait()
            pltpu.sync_copy(tok_hbm.at[pl.ds(rbase, _CH)], acc_v)

            for r in range(_CH):
                def dcol(v, c):
                    s = acc_v[r, pl.ds(v * 16, 16)]
                    for j in range(10):
                        s = s + rows_v[r * 10 + j, pl.ds(v * 16, 16)]
                    acc_v[r, pl.ds(v * 16, 16)] = s
                    return c
                lax.fori_loop(0, _D // 16, dcol, 0)

            pltpu.sync_copy(acc_v, out_hbm.at[pl.ds(rbase, _CH)])
            return carry

        lax.fori_loop(0, _RW // _CH, chunk, 0)

    return k(idsf, tok, tab)


def _tc_body(av_ref, gf_ref, gm_ref, gv_ref, sf_ref, sm_ref, sv_ref,
             vn_ref, gW_ref, gb_ref, sW_ref, sb_ref, out_ref):
    atom = av_ref[...]

    cls = jnp.broadcast_to(vn_ref[...], (_BB, _D))

    gf = gf_ref[...]
    gm = gm_ref[...]
    gv = gv_ref[...]
    gW = gW_ref[...]
    gb = gb_ref[...]
    glob = (gm[:, 0:1] * (gf[:, 0:1] * gW[0:1, :] + gb[0:1, :])
            + gm[:, 1:2] * (gf[:, 1:2] * gW[1:2, :] + gb[1:2, :]))
    glob = glob * gv

    sf = sf_ref[...]
    sm = sm_ref[...]
    sv = sv_ref[...]
    sW = sW_ref[...]
    sb = sb_ref[...]
    a0 = sv * sm[:, 0:1]
    a1 = sv * sm[:, 1:2]
    seg = ((a0 * sf[:, 0:1]) * sW[0:1, :] + a0 * sb[0:1, :]
           + (a1 * sf[:, 1:2]) * sW[1:2, :] + a1 * sb[1:2, :])

    out_ref[:, 0:1, :] = cls.reshape(_BB, 1, _D)
    out_ref[:, 1:2, :] = glob.reshape(_BB, 1, _D)
    out_ref[:, 2:2 + _S, :] = seg.reshape(_BB, _S, _D)
    out_ref[:, 2 + _S:, :] = atom.reshape(_BB, _N, _D)


def kernel(atom_feat, degree, segment_id, glob_feat, glob_mask, glob_valid_mask,
           seg_feat, seg_feat_mask, seg_valid_mask, token_feat, atom_table,
           degree_table, vnode, glob_W, glob_b, seg_W, seg_b):
    del segment_id
    dtype = token_feat.dtype
    idsf = jnp.concatenate(
        [atom_feat.reshape(_B * _N, _F), degree.reshape(_B * _N, 1) + _NA],
        axis=1).reshape(_B * _N * 10)
    tok = token_feat.reshape(_B * _N, _D)
    tab = jnp.concatenate([atom_table, degree_table], axis=0)

    atomvec = _sc_atom(idsf, tok, tab)

    sf = seg_feat.reshape(_B * _S, _NS)
    sm = seg_feat_mask.reshape(_B * _S, _NS)
    sv = seg_valid_mask.reshape(_B * _S, 1)

    grid = (_B // _BB,)
    out = pl.pallas_call(
        _tc_body,
        grid=grid,
        in_specs=[
            pl.BlockSpec((_R, _D), lambda i: (i, 0)),
            pl.BlockSpec((_BB, _NG), lambda i: (i, 0)),
            pl.BlockSpec((_BB, _NG), lambda i: (i, 0)),
            pl.BlockSpec((_BB, 1), lambda i: (i, 0)),
            pl.BlockSpec((_BB * _S, _NS), lambda i: (i, 0)),
            pl.BlockSpec((_BB * _S, _NS), lambda i: (i, 0)),
            pl.BlockSpec((_BB * _S, 1), lambda i: (i, 0)),
            pl.BlockSpec((1, _D), lambda i: (0, 0)),
            pl.BlockSpec((_NG, _D), lambda i: (0, 0)),
            pl.BlockSpec((_NG, _D), lambda i: (0, 0)),
            pl.BlockSpec((_NS, _D), lambda i: (0, 0)),
            pl.BlockSpec((_NS, _D), lambda i: (0, 0)),
        ],
        out_specs=pl.BlockSpec((_BB, 2 + _S + _N, _D), lambda i: (i, 0, 0)),
        out_shape=jax.ShapeDtypeStruct((_B, 2 + _S + _N, _D), dtype),
        compiler_params=pltpu.CompilerParams(
            dimension_semantics=("arbitrary",),
        ),
    )(atomvec, glob_feat, glob_mask, glob_valid_mask, sf, sm, sv,
      vnode, glob_W, glob_b, seg_W, seg_b)
    return out
